# baseline (speedup 1.0000x reference)
import functools

import jax
import jax.numpy as jnp
from jax import lax
from jax.experimental import pallas as pl
from jax.experimental.pallas import tpu as pltpu
from jax.experimental.pallas import tpu_sc as plsc

N = 1000000
H = 64
FD = 32
B = 16384

NC = 2
NS = 16
NW = NC * NS
TOT = 2 * B
ROWS_PER_W = TOT // NW
CH = 128
NCH = ROWS_PER_W // CH


def _sc_gather_kernel(mem_hbm, idx_hbm, out_hbm, idx_v, rows_v, sem):
    wid = lax.axis_index("s") * NC + lax.axis_index("c")
    base = wid * ROWS_PER_W
    pltpu.sync_copy(idx_hbm.at[pl.ds(base, ROWS_PER_W)], idx_v)
    copies = []
    for j in range(NCH):
        copies.append(pltpu.async_copy(
            mem_hbm.at[idx_v.at[pl.ds(j * CH, CH)]],
            rows_v.at[pl.ds(j * CH, CH)],
            sem,
        ))
    for c in copies:
        c.wait()
    pltpu.sync_copy(rows_v, out_hbm.at[pl.ds(base, ROWS_PER_W)])


def _sc_gather(mem, idx):
    mesh = plsc.VectorSubcoreMesh(core_axis_name="c", subcore_axis_name="s")
    return pl.kernel(
        _sc_gather_kernel,
        mesh=mesh,
        compiler_params=pltpu.CompilerParams(use_tc_tiling_on_sc=False),
        out_type=jax.ShapeDtypeStruct((TOT, H), jnp.float32),
        scratch_types=[
            pltpu.VMEM((ROWS_PER_W,), jnp.int32),
            pltpu.VMEM((ROWS_PER_W, H), jnp.float32),
            pltpu.SemaphoreType.DMA,
        ],
    )(mem, idx)


BLK = 2048
NBLK = B // BLK


def _tc_mlp_kernel(ms_ref, md_ref, f_ref, w1s_ref, w1d_ref, w1f_ref,
                   b1_ref, w2_ref, b2_ref, out_ref):
    acc = jnp.dot(ms_ref[:], w1s_ref[:], preferred_element_type=jnp.float32)
    acc += jnp.dot(md_ref[:], w1d_ref[:], preferred_element_type=jnp.float32)
    acc += jnp.dot(f_ref[:], w1f_ref[:], preferred_element_type=jnp.float32)
    h = jnp.maximum(acc + b1_ref[:], 0.0)
    out_ref[:] = jnp.sum(h * w2_ref[:], axis=1) + b2_ref[0, 0]


def _tc_mlp(gathered, feats, w1sT, w1dT, w1fT, b1, w2, b2):
    return pl.pallas_call(
        _tc_mlp_kernel,
        grid=(NBLK,),
        in_specs=[
            pl.BlockSpec((BLK, H), lambda i: (i, 0)),
            pl.BlockSpec((BLK, H), lambda i: (i + NBLK, 0)),
            pl.BlockSpec((BLK, FD), lambda i: (i, 0)),
            pl.BlockSpec((H, H), lambda i: (0, 0)),
            pl.BlockSpec((H, H), lambda i: (0, 0)),
            pl.BlockSpec((FD, H), lambda i: (0, 0)),
            pl.BlockSpec((1, H), lambda i: (0, 0)),
            pl.BlockSpec((1, H), lambda i: (0, 0)),
            pl.BlockSpec((1, 1), lambda i: (0, 0)),
        ],
        out_specs=pl.BlockSpec((BLK,), lambda i: (i,)),
        out_shape=jax.ShapeDtypeStruct((B,), jnp.float32),
    )(gathered, gathered, feats, w1sT, w1dT, w1fT, b1, w2, b2)


def kernel(s, d, feats, mem, w1, b1, w2, b2):
    idx = jnp.concatenate([s, d]).astype(jnp.int32)
    gathered = _sc_gather(mem, idx)
    w1sT = w1[:, :H].T
    w1dT = w1[:, H:2 * H].T
    w1fT = w1[:, 2 * H:].T
    return _tc_mlp(gathered, feats, w1sT, w1dT, w1fT,
                   b1.reshape(1, H), w2, b2.reshape(1, 1))


# speedup vs baseline: 1.0025x; 1.0025x over previous
import functools

import jax
import jax.numpy as jnp
from jax import lax
from jax.experimental import pallas as pl
from jax.experimental.pallas import tpu as pltpu
from jax.experimental.pallas import tpu_sc as plsc

N = 1000000
H = 64
FD = 32
B = 16384

NC = 2
NS = 16
NW = NC * NS
TOT = 2 * B
ROWS_PER_W = TOT // NW
CH = 256
NCH = ROWS_PER_W // CH


def _sc_gather_kernel(mem_hbm, idx_hbm, out_hbm, idx_v, buf0, buf1, gsem, osem):
    wid = lax.axis_index("s") * NC + lax.axis_index("c")
    base = wid * ROWS_PER_W
    pltpu.sync_copy(idx_hbm.at[pl.ds(base, ROWS_PER_W)], idx_v)
    bufs = (buf0, buf1)

    def gather(j):
        return pltpu.async_copy(
            mem_hbm.at[idx_v.at[pl.ds(j * CH, CH)]], bufs[j % 2], gsem)

    def outcopy(j):
        return pltpu.async_copy(
            bufs[j % 2], out_hbm.at[pl.ds(base + j * CH, CH)], osem)

    gc = {0: gather(0), 1: gather(1)}
    oc = {}
    for j in range(NCH):
        gc[j].wait()
        oc[j] = outcopy(j)
        nj = j + 2
        if nj < NCH:
            oc[nj - 2].wait()
            gc[nj] = gather(nj)
    oc[NCH - 2].wait()
    oc[NCH - 1].wait()


def _sc_gather(mem128, idx):
    mesh = plsc.VectorSubcoreMesh(core_axis_name="c", subcore_axis_name="s")
    return pl.kernel(
        _sc_gather_kernel,
        mesh=mesh,
        out_type=jax.ShapeDtypeStruct((TOT, 2 * H), jnp.float32),
        scratch_types=[
            pltpu.VMEM((ROWS_PER_W,), jnp.int32),
            pltpu.VMEM((CH, 2 * H), jnp.float32),
            pltpu.VMEM((CH, 2 * H), jnp.float32),
            pltpu.SemaphoreType.DMA,
            pltpu.SemaphoreType.DMA,
        ],
    )(mem128, idx)


BLK = 2048
NBLK = B // BLK


def _tc_mlp_kernel(ms_ref, md_ref, f_ref, ps_ref, pd_ref,
                   w1s_ref, w1d_ref, w1f_ref, b1_ref, w2_ref, b2_ref, out_ref):
    lanes = lax.broadcasted_iota(jnp.int32, (BLK, 2 * H), 1)
    hi = lanes >= H
    mask_s = jnp.where(hi == (ps_ref[:] != 0), 1.0, 0.0)
    mask_d = jnp.where(hi == (pd_ref[:] != 0), 1.0, 0.0)
    acc = jnp.dot(ms_ref[:] * mask_s, w1s_ref[:],
                  preferred_element_type=jnp.float32)
    acc += jnp.dot(md_ref[:] * mask_d, w1d_ref[:],
                   preferred_element_type=jnp.float32)
    acc += jnp.dot(f_ref[:], w1f_ref[:], preferred_element_type=jnp.float32)
    h = jnp.maximum(acc + b1_ref[:], 0.0)
    out_ref[:] = jnp.sum(h * w2_ref[:], axis=1) + b2_ref[0, 0]


def _tc_mlp(gathered, feats, ps, pd, w1s2, w1d2, w1fT, b1, w2, b2):
    return pl.pallas_call(
        _tc_mlp_kernel,
        grid=(NBLK,),
        in_specs=[
            pl.BlockSpec((BLK, 2 * H), lambda i: (i, 0)),
            pl.BlockSpec((BLK, 2 * H), lambda i: (i + NBLK, 0)),
            pl.BlockSpec((BLK, FD), lambda i: (i, 0)),
            pl.BlockSpec((BLK, 1), lambda i: (i, 0)),
            pl.BlockSpec((BLK, 1), lambda i: (i, 0)),
            pl.BlockSpec((2 * H, H), lambda i: (0, 0)),
            pl.BlockSpec((2 * H, H), lambda i: (0, 0)),
            pl.BlockSpec((FD, H), lambda i: (0, 0)),
            pl.BlockSpec((1, H), lambda i: (0, 0)),
            pl.BlockSpec((1, H), lambda i: (0, 0)),
            pl.BlockSpec((1, 1), lambda i: (0, 0)),
        ],
        out_specs=pl.BlockSpec((BLK,), lambda i: (i,)),
        out_shape=jax.ShapeDtypeStruct((B,), jnp.float32),
    )(gathered, gathered, feats, ps, pd, w1s2, w1d2, w1fT, b1, w2, b2)


def kernel(s, d, feats, mem, w1, b1, w2, b2):
    s = s.astype(jnp.int32)
    d = d.astype(jnp.int32)
    idx = jnp.concatenate([s, d]) >> 1
    ps = (s & 1).reshape(B, 1)
    pd = (d & 1).reshape(B, 1)
    mem128 = mem.reshape(N // 2, 2 * H)
    gathered = _sc_gather(mem128, idx)
    w1sT = w1[:, :H].T
    w1dT = w1[:, H:2 * H].T
    w1fT = w1[:, 2 * H:].T
    w1s2 = jnp.concatenate([w1sT, w1sT], axis=0)
    w1d2 = jnp.concatenate([w1dT, w1dT], axis=0)
    return _tc_mlp(gathered, feats, ps, pd, w1s2, w1d2, w1fT,
                   b1.reshape(1, H), w2, b2.reshape(1, 1))
